# Initial kernel scaffold; baseline (speedup 1.0000x reference)
import jax
import jax.numpy as jnp
from jax.experimental import pallas as pl


def kernel(x, edge_index, batch, W1, b1, W2, b2, W3, b3, W4, b4, Wfc, bfc):
    raise NotImplementedError("write your pallas kernel here")


# speedup vs baseline: 68.8250x; 68.8250x over previous
import functools

import jax
import jax.numpy as jnp
from jax import lax
from jax.experimental import pallas as pl
from jax.experimental.pallas import tpu as pltpu
from jax.experimental.pallas import tpu_sc as plsc

N = 100000
NP = 100352
E = 3200000
G = 64
F = 8

NC = 2
NS = 16
NW = NC * NS
L = 128
NR = E // L
RB = 40
NG = NR // RB
GPW = -(-NG // NW)
U = 8
CZ = 2048
NZ = NP // CZ

_mesh = plsc.VectorSubcoreMesh(core_axis_name="c", subcore_axis_name="s")
_sc_params = pltpu.CompilerParams(use_tc_tiling_on_sc=False)


@functools.partial(
    pl.kernel,
    out_type=jax.ShapeDtypeStruct((NC, NP, 1), jnp.float32),
    mesh=_mesh,
    scratch_types=[
        pltpu.VMEM((RB, L), jnp.int32),
        pltpu.VMEM((RB, L), jnp.int32),
        pltpu.VMEM((RB, L, 1), jnp.float32),
        pltpu.VMEM_SHARED((NP, 1), jnp.float32),
        pltpu.VMEM_SHARED((NP, 1), jnp.float32),
        pltpu.SemaphoreType.DMA,
        pltpu.SemaphoreType.DMA,
    ],
    compiler_params=_sc_params,
)
def _sc_agg1(x_hbm, e3_hbm, z_hbm, out_hbm, srcv, dstv, valv, xt_sh, agg_sh,
             gsem, ssem):
    cid = lax.axis_index("c")
    sid = lax.axis_index("s")
    w = cid * NS + sid

    @pl.loop(0, 4)
    def _(j):
        cc = sid + NS * j

        @pl.when(cc < NZ)
        def _():
            sl = pl.ds(cc * CZ, CZ)
            pltpu.sync_copy(x_hbm.at[sl], xt_sh.at[sl])
            pltpu.sync_copy(z_hbm.at[sl], agg_sh.at[sl])

    plsc.subcore_barrier()

    @pl.loop(0, GPW)
    def _(t):
        g = w + NW * t

        @pl.when(g < NG)
        def _():
            row0 = g * RB
            pltpu.sync_copy(e3_hbm.at[0, pl.ds(row0, RB)], srcv)
            pltpu.sync_copy(e3_hbm.at[1, pl.ds(row0, RB)], dstv)

            @pl.loop(0, RB, step=U)
            def _(r0):
                gs = [
                    pltpu.async_copy(
                        xt_sh.at[srcv.at[r0 + i]], valv.at[r0 + i], gsem
                    )
                    for i in range(U)
                ]
                for cp in gs:
                    cp.wait()
                ss = [
                    pltpu.async_copy(
                        valv.at[r0 + i], agg_sh.at[dstv.at[r0 + i]], ssem,
                        add=True,
                    )
                    for i in range(U)
                ]
                for cp in ss:
                    cp.wait()

    plsc.subcore_barrier()

    @pl.when(sid == 0)
    def _():
        pltpu.sync_copy(agg_sh, out_hbm.at[cid])


@functools.partial(
    pl.kernel,
    out_type=jax.ShapeDtypeStruct((NC, NP, F), jnp.float32),
    mesh=_mesh,
    scratch_types=[
        pltpu.VMEM((RB, L), jnp.int32),
        pltpu.VMEM((RB, L), jnp.int32),
        pltpu.VMEM((U, L, F), jnp.float32),
        pltpu.VMEM_SHARED((NP, F), jnp.float32),
        pltpu.VMEM_SHARED((NP, F), jnp.float32),
        pltpu.SemaphoreType.DMA,
        pltpu.SemaphoreType.DMA,
    ],
    compiler_params=_sc_params,
)
def _sc_agg2(h_hbm, e3_hbm, z_hbm, out_hbm, srcv, dstv, valv, ht_sh, agg_sh,
             gsem, ssem):
    cid = lax.axis_index("c")
    sid = lax.axis_index("s")
    w = cid * NS + sid

    @pl.loop(0, 4)
    def _(j):
        cc = sid + NS * j

        @pl.when(cc < NZ)
        def _():
            sl = pl.ds(cc * CZ, CZ)
            pltpu.sync_copy(h_hbm.at[sl], ht_sh.at[sl])
            pltpu.sync_copy(z_hbm.at[sl], agg_sh.at[sl])

    plsc.subcore_barrier()

    @pl.loop(0, GPW)
    def _(t):
        g = w + NW * t

        @pl.when(g < NG)
        def _():
            row0 = g * RB
            pltpu.sync_copy(e3_hbm.at[0, pl.ds(row0, RB)], srcv)
            pltpu.sync_copy(e3_hbm.at[1, pl.ds(row0, RB)], dstv)

            @pl.loop(0, RB, step=U)
            def _(r0):
                gs = [
                    pltpu.async_copy(
                        ht_sh.at[srcv.at[r0 + i]], valv.at[i], gsem
                    )
                    for i in range(U)
                ]
                for cp in gs:
                    cp.wait()
                ss = [
                    pltpu.async_copy(
                        valv.at[i], agg_sh.at[dstv.at[r0 + i]], ssem,
                        add=True,
                    )
                    for i in range(U)
                ]
                for cp in ss:
                    cp.wait()

    plsc.subcore_barrier()

    @pl.when(sid == 0)
    def _():
        pltpu.sync_copy(agg_sh, out_hbm.at[cid])


B1 = 3136


def _tc1_body(x_ref, p_ref, w1_ref, b1_ref, w2_ref, b2_ref, h_ref):
    s = x_ref[...] + p_ref[0] + p_ref[1]
    hid = jnp.maximum(s * w1_ref[...] + b1_ref[...], 0.0)
    h = (
        jnp.dot(hid, w2_ref[...], preferred_element_type=jnp.float32)
        + b2_ref[...]
    )
    h_ref[...] = jnp.where(h > 0, h, jnp.exp(h) - 1.0)


def _tc1(x2, p, W1, b1, W2, b2):
    return pl.pallas_call(
        _tc1_body,
        grid=(NP // B1,),
        in_specs=[
            pl.BlockSpec((B1, 1), lambda j: (j, 0)),
            pl.BlockSpec((NC, B1, 1), lambda j: (0, j, 0)),
            pl.BlockSpec((1, F), lambda j: (0, 0)),
            pl.BlockSpec((1, F), lambda j: (0, 0)),
            pl.BlockSpec((F, F), lambda j: (0, 0)),
            pl.BlockSpec((1, F), lambda j: (0, 0)),
        ],
        out_specs=pl.BlockSpec((B1, F), lambda j: (j, 0)),
        out_shape=jax.ShapeDtypeStruct((NP, F), jnp.float32),
    )(x2, p, W1, b1, W2, b2)


def _tc2_body(h_ref, q_ref, batch_ref, w3_ref, b3_ref, w4_ref, b4_ref,
              wfct_ref, bfc_ref, out_ref, acc_ref, cnt_ref):
    j = pl.program_id(0)

    @pl.when(j == 0)
    def _():
        acc_ref[...] = jnp.zeros_like(acc_ref)
        cnt_ref[...] = jnp.zeros_like(cnt_ref)

    u = h_ref[...] + q_ref[0] + q_ref[1]
    z = jnp.maximum(
        jnp.dot(u, w3_ref[...], preferred_element_type=jnp.float32)
        + b3_ref[...],
        0.0,
    )
    p2 = (
        jnp.dot(z, w4_ref[...], preferred_element_type=jnp.float32)
        + b4_ref[...]
    )
    oh = (
        batch_ref[...] == lax.broadcasted_iota(jnp.int32, (B1, G), 1)
    ).astype(jnp.float32)
    acc_ref[...] += lax.dot_general(
        p2, oh, (((0,), (0,)), ((), ())), preferred_element_type=jnp.float32
    )
    cnt_ref[...] += jnp.sum(oh, axis=0)[None, :]

    @pl.when(j == pl.num_programs(0) - 1)
    def _():
        pooled = acc_ref[...] / jnp.maximum(cnt_ref[...], 1.0)
        logit = (
            jnp.dot(wfct_ref[...], pooled, preferred_element_type=jnp.float32)
            + bfc_ref[...]
        )
        out_ref[...] = jax.nn.sigmoid(logit)


def _tc2(h8, q, batch2, W3, b3, W4, b4, WfcT, bfc):
    return pl.pallas_call(
        _tc2_body,
        grid=(NP // B1,),
        in_specs=[
            pl.BlockSpec((B1, F), lambda j: (j, 0)),
            pl.BlockSpec((NC, B1, F), lambda j: (0, j, 0)),
            pl.BlockSpec((B1, 1), lambda j: (j, 0)),
            pl.BlockSpec((F, F), lambda j: (0, 0)),
            pl.BlockSpec((1, F), lambda j: (0, 0)),
            pl.BlockSpec((F, F), lambda j: (0, 0)),
            pl.BlockSpec((1, F), lambda j: (0, 0)),
            pl.BlockSpec((1, F), lambda j: (0, 0)),
            pl.BlockSpec((1, 1), lambda j: (0, 0)),
        ],
        out_specs=pl.BlockSpec((1, G), lambda j: (0, 0)),
        out_shape=jax.ShapeDtypeStruct((1, G), jnp.float32),
        scratch_shapes=[
            pltpu.VMEM((F, G), jnp.float32),
            pltpu.VMEM((1, G), jnp.float32),
        ],
    )(h8, q, batch2, W3, b3, W4, b4, WfcT, bfc)


def kernel(x, edge_index, batch, W1, b1, W2, b2, W3, b3, W4, b4, Wfc, bfc):
    xf = jnp.pad(x.reshape(-1), (0, NP - N))
    batch2 = jnp.pad(batch, (0, NP - N), constant_values=G).reshape(NP, 1)
    e3 = edge_index.reshape(2, NR, L)
    z1 = jnp.zeros((NP, 1), jnp.float32)
    z8 = jnp.zeros((NP, F), jnp.float32)
    p = _sc_agg1(xf.reshape(NP, 1), e3, z1)
    h8 = _tc1(
        xf.reshape(NP, 1), p,
        W1, b1.reshape(1, F), W2, b2.reshape(1, F),
    )
    q = _sc_agg2(h8, e3, z8)
    out = _tc2(
        h8, q, batch2,
        W3, b3.reshape(1, F), W4, b4.reshape(1, F),
        Wfc.T, bfc.reshape(1, 1),
    )
    return out.reshape(-1)


# speedup vs baseline: 90.3301x; 1.3125x over previous
import functools

import jax
import jax.numpy as jnp
from jax import lax
from jax.experimental import pallas as pl
from jax.experimental.pallas import tpu as pltpu
from jax.experimental.pallas import tpu_sc as plsc

N = 100000
NP = 100352
E = 3200000
G = 64
F = 8

NC = 2
NS = 16
NW = NC * NS
L = 128
NR = E // L
RB = 40
NG = NR // RB
GPW = -(-NG // NW)
U = 8
CZ = 2048
NZ = NP // CZ

_mesh = plsc.VectorSubcoreMesh(core_axis_name="c", subcore_axis_name="s")
_sc_params = pltpu.CompilerParams(use_tc_tiling_on_sc=False,
                                  internal_scratch_in_bytes=131072)


def _sc_agg_kernel(table_hbm, e3_hbm, out_hbm, srcv, dstv, valv, tab_sh,
                   agg_sh, gsem, ssem, isem, u):
    cid = lax.axis_index("c")
    sid = lax.axis_index("s")
    w = cid * NS + sid

    @pl.loop(0, 4)
    def _(j):
        cc = sid + NS * j

        @pl.when(cc < NZ)
        def _():
            sl = pl.ds(cc * CZ, CZ)
            pltpu.sync_copy(table_hbm.at[sl], tab_sh.at[sl])
            pltpu.sync_copy(table_hbm.at[sl], agg_sh.at[sl])

    pltpu.async_copy(e3_hbm.at[0, pl.ds(w * RB, RB)], srcv.at[0], isem)
    pltpu.async_copy(e3_hbm.at[1, pl.ds(w * RB, RB)], dstv.at[0], isem)

    plsc.subcore_barrier()

    @pl.loop(0, GPW)
    def _(t):
        g = w + NW * t

        @pl.when(g < NG)
        def _():
            p = lax.rem(t, 2)
            pltpu.make_async_copy(
                e3_hbm.at[0, pl.ds(0, RB)], srcv.at[p], isem).wait()
            pltpu.make_async_copy(
                e3_hbm.at[1, pl.ds(0, RB)], dstv.at[p], isem).wait()
            gn = g + NW

            @pl.when(gn < NG)
            def _():
                pn = lax.rem(t + 1, 2)
                pltpu.async_copy(
                    e3_hbm.at[0, pl.ds(gn * RB, RB)], srcv.at[pn], isem)
                pltpu.async_copy(
                    e3_hbm.at[1, pl.ds(gn * RB, RB)], dstv.at[pn], isem)

            @pl.loop(0, RB, step=u)
            def _(r0):
                gs = [
                    pltpu.async_copy(
                        tab_sh.at[srcv.at[p, r0 + i]], valv.at[i],
                        gsem.at[i],
                    )
                    for i in range(u)
                ]
                ss = []
                for i in range(u):
                    gs[i].wait()
                    ss.append(pltpu.async_copy(
                        valv.at[i], agg_sh.at[dstv.at[p, r0 + i]], ssem,
                        add=True,
                    ))
                for cp in ss:
                    cp.wait()

    plsc.subcore_barrier()

    @pl.when(sid == 0)
    def _():
        pltpu.sync_copy(agg_sh, out_hbm.at[cid])


@functools.partial(
    pl.kernel,
    out_type=jax.ShapeDtypeStruct((NC, NP, 1), jnp.float32),
    mesh=_mesh,
    scratch_types=[
        pltpu.VMEM((2, RB, L), jnp.int32),
        pltpu.VMEM((2, RB, L), jnp.int32),
        pltpu.VMEM((U, L, 1), jnp.float32),
        pltpu.VMEM_SHARED((NP, 1), jnp.float32),
        pltpu.VMEM_SHARED((NP, 1), jnp.float32),
        pltpu.SemaphoreType.DMA((U,)),
        pltpu.SemaphoreType.DMA,
        pltpu.SemaphoreType.DMA,
    ],
    compiler_params=_sc_params,
)
def _sc_agg1(x_hbm, e3_hbm, out_hbm, srcv, dstv, valv, xt_sh, agg_sh,
             gsem, ssem, isem):
    _sc_agg_kernel(x_hbm, e3_hbm, out_hbm, srcv, dstv, valv, xt_sh, agg_sh,
                   gsem, ssem, isem, U)


@functools.partial(
    pl.kernel,
    out_type=jax.ShapeDtypeStruct((NC, NP, F), jnp.float32),
    mesh=_mesh,
    scratch_types=[
        pltpu.VMEM((2, RB, L), jnp.int32),
        pltpu.VMEM((2, RB, L), jnp.int32),
        pltpu.VMEM((U, L, F), jnp.float32),
        pltpu.VMEM_SHARED((NP, F), jnp.float32),
        pltpu.VMEM_SHARED((NP, F), jnp.float32),
        pltpu.SemaphoreType.DMA((U,)),
        pltpu.SemaphoreType.DMA,
        pltpu.SemaphoreType.DMA,
    ],
    compiler_params=_sc_params,
)
def _sc_agg2(h_hbm, e3_hbm, out_hbm, srcv, dstv, valv, ht_sh, agg_sh,
             gsem, ssem, isem):
    _sc_agg_kernel(h_hbm, e3_hbm, out_hbm, srcv, dstv, valv, ht_sh, agg_sh,
                   gsem, ssem, isem, U)


B1 = 3136


def _tc1_body(x_ref, p_ref, w1_ref, b1_ref, w2_ref, b2_ref, h_ref):
    s = p_ref[0] + p_ref[1] - x_ref[...]
    hid = jnp.maximum(s * w1_ref[...] + b1_ref[...], 0.0)
    h = (
        jnp.dot(hid, w2_ref[...], preferred_element_type=jnp.float32)
        + b2_ref[...]
    )
    h_ref[...] = jnp.where(h > 0, h, jnp.exp(h) - 1.0)


def _tc1(x2, p, W1, b1, W2, b2):
    return pl.pallas_call(
        _tc1_body,
        grid=(NP // B1,),
        in_specs=[
            pl.BlockSpec((B1, 1), lambda j: (j, 0)),
            pl.BlockSpec((NC, B1, 1), lambda j: (0, j, 0)),
            pl.BlockSpec((1, F), lambda j: (0, 0)),
            pl.BlockSpec((1, F), lambda j: (0, 0)),
            pl.BlockSpec((F, F), lambda j: (0, 0)),
            pl.BlockSpec((1, F), lambda j: (0, 0)),
        ],
        out_specs=pl.BlockSpec((B1, F), lambda j: (j, 0)),
        out_shape=jax.ShapeDtypeStruct((NP, F), jnp.float32),
    )(x2, p, W1, b1, W2, b2)


def _tc2_body(h_ref, q_ref, batch_ref, w3_ref, b3_ref, w4_ref, b4_ref,
              wfct_ref, bfc_ref, out_ref, acc_ref, cnt_ref):
    j = pl.program_id(0)

    @pl.when(j == 0)
    def _():
        acc_ref[...] = jnp.zeros_like(acc_ref)
        cnt_ref[...] = jnp.zeros_like(cnt_ref)

    u = q_ref[0] + q_ref[1] - h_ref[...]
    z = jnp.maximum(
        jnp.dot(u, w3_ref[...], preferred_element_type=jnp.float32)
        + b3_ref[...],
        0.0,
    )
    p2 = (
        jnp.dot(z, w4_ref[...], preferred_element_type=jnp.float32)
        + b4_ref[...]
    )
    oh = (
        batch_ref[...] == lax.broadcasted_iota(jnp.int32, (B1, G), 1)
    ).astype(jnp.float32)
    acc_ref[...] += lax.dot_general(
        p2, oh, (((0,), (0,)), ((), ())), preferred_element_type=jnp.float32
    )
    cnt_ref[...] += jnp.sum(oh, axis=0)[None, :]

    @pl.when(j == pl.num_programs(0) - 1)
    def _():
        pooled = acc_ref[...] / jnp.maximum(cnt_ref[...], 1.0)
        logit = (
            jnp.dot(wfct_ref[...], pooled, preferred_element_type=jnp.float32)
            + bfc_ref[...]
        )
        out_ref[...] = jax.nn.sigmoid(logit)


def _tc2(h8, q, batch2, W3, b3, W4, b4, WfcT, bfc):
    return pl.pallas_call(
        _tc2_body,
        grid=(NP // B1,),
        in_specs=[
            pl.BlockSpec((B1, F), lambda j: (j, 0)),
            pl.BlockSpec((NC, B1, F), lambda j: (0, j, 0)),
            pl.BlockSpec((B1, 1), lambda j: (j, 0)),
            pl.BlockSpec((F, F), lambda j: (0, 0)),
            pl.BlockSpec((1, F), lambda j: (0, 0)),
            pl.BlockSpec((F, F), lambda j: (0, 0)),
            pl.BlockSpec((1, F), lambda j: (0, 0)),
            pl.BlockSpec((1, F), lambda j: (0, 0)),
            pl.BlockSpec((1, 1), lambda j: (0, 0)),
        ],
        out_specs=pl.BlockSpec((1, G), lambda j: (0, 0)),
        out_shape=jax.ShapeDtypeStruct((1, G), jnp.float32),
        scratch_shapes=[
            pltpu.VMEM((F, G), jnp.float32),
            pltpu.VMEM((1, G), jnp.float32),
        ],
    )(h8, q, batch2, W3, b3, W4, b4, WfcT, bfc)


def kernel(x, edge_index, batch, W1, b1, W2, b2, W3, b3, W4, b4, Wfc, bfc):
    xf = jnp.pad(x.reshape(-1), (0, NP - N))
    batch2 = jnp.pad(batch, (0, NP - N), constant_values=G).reshape(NP, 1)
    e3 = edge_index.reshape(2, NR, L)
    p = _sc_agg1(xf.reshape(NP, 1), e3)
    h8 = _tc1(
        xf.reshape(NP, 1), p,
        W1, b1.reshape(1, F), W2, b2.reshape(1, F),
    )
    q = _sc_agg2(h8, e3)
    out = _tc2(
        h8, q, batch2,
        W3, b3.reshape(1, F), W4, b4.reshape(1, F),
        Wfc.T, bfc.reshape(1, 1),
    )
    return out.reshape(-1)
